# baseline (speedup 1.0000x reference)
import functools

import jax
import jax.numpy as jnp
from jax import lax
from jax.experimental import pallas as pl
from jax.experimental.pallas import tpu as pltpu
from jax.experimental.pallas import tpu_sc as plsc

NUM_LABELS = 1000000
EMBED_DIM = 16
BATCH = 16384

_NC = 2
_NS = 16
_NW = _NC * _NS
_PER_W = BATCH // _NW
_CHUNK = 128
_NCHUNK = _PER_W // _CHUNK


def _gather_kernel(idx_hbm, table_hbm, out_hbm, idx_v, rows_v, sem):
    wid = lax.axis_index("s") * _NC + lax.axis_index("c")
    base = wid * _PER_W
    pltpu.sync_copy(idx_hbm.at[pl.ds(base, _PER_W)], idx_v)

    def body(j, carry):
        off = j * _CHUNK
        copies = [
            pltpu.async_copy(
                table_hbm.at[d].at[idx_v.at[pl.ds(off, _CHUNK)]],
                rows_v.at[d, pl.ds(off, _CHUNK)],
                sem,
            )
            for d in range(EMBED_DIM)
        ]
        for c in copies:
            c.wait()
        return carry

    lax.fori_loop(0, _NCHUNK, body, 0)
    pltpu.sync_copy(rows_v, out_hbm.at[:, pl.ds(base, _PER_W)])


@jax.jit
def kernel(labels, label_embs):
    idx = labels.astype(jnp.int32)
    table_t = label_embs.T
    mesh = plsc.VectorSubcoreMesh(core_axis_name="c", subcore_axis_name="s")
    call = functools.partial(
        pl.kernel,
        mesh=mesh,
        out_type=jax.ShapeDtypeStruct((EMBED_DIM, BATCH), jnp.float32),
        scratch_types=[
            pltpu.VMEM((_PER_W,), jnp.int32),
            pltpu.VMEM((EMBED_DIM, _PER_W), jnp.float32),
            pltpu.SemaphoreType.DMA,
        ],
        compiler_params=pltpu.CompilerParams(use_tc_tiling_on_sc=False),
    )(_gather_kernel)
    out_t = call(idx, table_t)
    return out_t.T


# speedup vs baseline: 7.5193x; 7.5193x over previous
import functools

import jax
import jax.numpy as jnp
from jax import lax
from jax.experimental import pallas as pl
from jax.experimental.pallas import tpu as pltpu
from jax.experimental.pallas import tpu_sc as plsc

NUM_LABELS = 1000000
EMBED_DIM = 16
BATCH = 16384
VOCAB = NUM_LABELS + 1

_NC = 2
_NS = 16
_NW = _NC * _NS
_L = 16

_W1 = 512
_MAIN = (VOCAB // _W1) * _W1
_NCOL = _MAIN // _W1
_NCHUNKS1 = 2 * _NCOL
_ITERS1 = -(-_NCHUNKS1 // _NW)
_RSTRIDE = 1000064
_FLAT = EMBED_DIM * _RSTRIDE
_TAILW = VOCAB - _MAIN

_PER_W = BATCH // _NW
_CHUNK = 128
_NCHUNK = _PER_W // _CHUNK
_NGROUP = _PER_W // _L


def _detile_kernel(table_hbm, tail_hbm, flat_hbm, buf_v, tail_v, sem):
    wid = lax.axis_index("s") * _NC + lax.axis_index("c")

    @pl.when(wid < 2)
    def _():
        a = wid
        pltpu.sync_copy(tail_hbm.at[pl.ds(a * 8, 8), :], tail_v)
        tail_copies = [
            pltpu.async_copy(
                tail_v.at[s],
                flat_hbm.at[pl.ds((a * 8 + s) * _RSTRIDE + _MAIN, _TAILW)],
                sem,
            )
            for s in range(8)
        ]
        for c in tail_copies:
            c.wait()

    def body(i, carry):
        k = wid + i * _NW

        @pl.when(k < _NCHUNKS1)
        def _():
            a = k % 2
            c0 = (k // 2) * _W1
            pltpu.sync_copy(
                table_hbm.at[pl.ds(a * 8, 8), pl.ds(c0, _W1)], buf_v
            )
            copies = [
                pltpu.async_copy(
                    buf_v.at[s],
                    flat_hbm.at[pl.ds((a * 8 + s) * _RSTRIDE + c0, _W1)],
                    sem,
                )
                for s in range(8)
            ]
            for c in copies:
                c.wait()

        return carry

    lax.fori_loop(0, _ITERS1, body, 0)


def _gather_kernel(idx_hbm, flat_hbm, out_hbm, idx_v, rows_v, sem):
    wid = lax.axis_index("s") * _NC + lax.axis_index("c")
    base = wid * _PER_W
    pltpu.sync_copy(idx_hbm.at[pl.ds(base, _PER_W)], idx_v)

    def body(j, carry):
        off = j * _CHUNK
        copies = [
            pltpu.async_copy(
                flat_hbm.at[pl.ds(d * _RSTRIDE, _RSTRIDE)]
                .at[idx_v.at[pl.ds(off, _CHUNK)]],
                rows_v.at[d, pl.ds(off, _CHUNK)],
                sem,
            )
            for d in range(EMBED_DIM)
        ]
        for c in copies:
            c.wait()
        return carry

    lax.fori_loop(0, _NCHUNK, body, 0)
    pltpu.sync_copy(rows_v, out_hbm.at[:, pl.ds(base, _PER_W)])


@jax.jit
def kernel(labels, label_embs):
    idx = labels.astype(jnp.int32)
    table_t = label_embs.T
    tail = label_embs[_MAIN:, :].T
    mesh = plsc.VectorSubcoreMesh(core_axis_name="c", subcore_axis_name="s")

    detile = functools.partial(
        pl.kernel,
        mesh=mesh,
        out_type=jax.ShapeDtypeStruct((_FLAT,), jnp.float32),
        scratch_types=[
            pltpu.VMEM((8, _W1), jnp.float32),
            pltpu.VMEM((8, _TAILW), jnp.float32),
            pltpu.SemaphoreType.DMA,
        ],
    )(_detile_kernel)
    flat = detile(table_t, tail)

    gather = functools.partial(
        pl.kernel,
        mesh=mesh,
        out_type=jax.ShapeDtypeStruct((EMBED_DIM, BATCH), jnp.float32),
        scratch_types=[
            pltpu.VMEM((_PER_W,), jnp.int32),
            pltpu.VMEM((EMBED_DIM, _PER_W), jnp.float32),
            pltpu.SemaphoreType.DMA,
        ],
        compiler_params=pltpu.CompilerParams(use_tc_tiling_on_sc=False),
    )(_gather_kernel)
    out_t = gather(idx, flat)
    return out_t.T


# speedup vs baseline: 13.4629x; 1.7905x over previous
import functools

import jax
import jax.numpy as jnp
from jax import lax
from jax.experimental import pallas as pl
from jax.experimental.pallas import tpu as pltpu
from jax.experimental.pallas import tpu_sc as plsc

NUM_LABELS = 1000000
EMBED_DIM = 16
BATCH = 16384
VOCAB = NUM_LABELS + 1

_NC = 2
_NS = 16
_NW = _NC * _NS
_L = 16

_W1 = 4608
_MAIN = 999936
_NCOL = _MAIN // _W1
_NCHUNKS1 = 2 * _NCOL
_ITERS1 = -(-_NCHUNKS1 // _NW)
_RSTRIDE = 1000064
_FLAT = EMBED_DIM * _RSTRIDE
_TAILW = VOCAB - _MAIN

_PER_W = BATCH // _NW
_CHUNK = 128
_NCHUNK = _PER_W // _CHUNK
_NGROUP = _PER_W // _L


def _detile_kernel(table_hbm, tail_hbm, flat_hbm, buf_a, buf_b, tail_v,
                   in_sem, out_sem, out_sem_b):
    out_sems = (out_sem, out_sem_b)
    wid = lax.axis_index("s") * _NC + lax.axis_index("c")
    bufs = (buf_a, buf_b)

    @pl.when(wid < 2)
    def _():
        a = wid
        pltpu.sync_copy(tail_hbm.at[pl.ds(a * 8, 8), :], tail_v)
        tail_copies = [
            pltpu.async_copy(
                tail_v.at[s],
                flat_hbm.at[pl.ds((a * 8 + s) * _RSTRIDE + _MAIN, _TAILW)],
                out_sem,
            )
            for s in range(8)
        ]
        for c in tail_copies:
            c.wait()

    def chunk(i):
        k = jnp.minimum(wid + i * _NW, _NCHUNKS1 - 1)
        return k % 2, (k // 2) * _W1

    def start_in(i):
        a, c0 = chunk(i)
        return pltpu.async_copy(
            table_hbm.at[pl.ds(a * 8, 8), pl.ds(c0, _W1)],
            bufs[i % 2],
            in_sem,
        )

    def start_outs(i):
        a, c0 = chunk(i)
        return [
            pltpu.async_copy(
                bufs[i % 2].at[s],
                flat_hbm.at[pl.ds((a * 8 + s) * _RSTRIDE + c0, _W1)],
                out_sems[i % 2],
            )
            for s in range(8)
        ]

    in_d = start_in(0)
    outs_prev = None
    for i in range(_ITERS1):
        in_d.wait()
        outs_i = start_outs(i)
        if i + 1 < _ITERS1:
            if outs_prev is not None:
                for c in outs_prev:
                    c.wait()
            in_d = start_in(i + 1)
            outs_prev = outs_i
        else:
            if outs_prev is not None:
                for c in outs_prev:
                    c.wait()
            for c in outs_i:
                c.wait()


def _gather_kernel(idx_hbm, flat_hbm, out_hbm, idx_v, rows_v, sem):
    wid = lax.axis_index("s") * _NC + lax.axis_index("c")
    base = wid * _PER_W
    pltpu.sync_copy(idx_hbm.at[pl.ds(base, _PER_W)], idx_v)

    def body(j, carry):
        off = j * _CHUNK
        copies = [
            pltpu.async_copy(
                flat_hbm.at[pl.ds(d * _RSTRIDE, _RSTRIDE)]
                .at[idx_v.at[pl.ds(off, _CHUNK)]],
                rows_v.at[d, pl.ds(off, _CHUNK)],
                sem,
            )
            for d in range(EMBED_DIM)
        ]
        for c in copies:
            c.wait()
        return carry

    lax.fori_loop(0, _NCHUNK, body, 0)
    pltpu.sync_copy(rows_v, out_hbm.at[:, pl.ds(base, _PER_W)])


@jax.jit
def kernel(labels, label_embs):
    idx = labels.astype(jnp.int32)
    table_t = label_embs.T
    tail = label_embs[_MAIN:, :].T
    mesh = plsc.VectorSubcoreMesh(core_axis_name="c", subcore_axis_name="s")

    detile = functools.partial(
        pl.kernel,
        mesh=mesh,
        out_type=jax.ShapeDtypeStruct((_FLAT,), jnp.float32),
        scratch_types=[
            pltpu.VMEM((8, _W1), jnp.float32),
            pltpu.VMEM((8, _W1), jnp.float32),
            pltpu.VMEM((8, _TAILW), jnp.float32),
            pltpu.SemaphoreType.DMA,
            pltpu.SemaphoreType.DMA,
            pltpu.SemaphoreType.DMA,
        ],
    )(_detile_kernel)
    flat = detile(table_t, tail)

    gather = functools.partial(
        pl.kernel,
        mesh=mesh,
        out_type=jax.ShapeDtypeStruct((EMBED_DIM, BATCH), jnp.float32),
        scratch_types=[
            pltpu.VMEM((_PER_W,), jnp.int32),
            pltpu.VMEM((EMBED_DIM, _PER_W), jnp.float32),
            pltpu.SemaphoreType.DMA,
        ],
        compiler_params=pltpu.CompilerParams(use_tc_tiling_on_sc=False),
    )(_gather_kernel)
    out_t = gather(idx, flat)
    return out_t.T


# speedup vs baseline: 13.4738x; 1.0008x over previous
import functools

import jax
import jax.numpy as jnp
from jax import lax
from jax.experimental import pallas as pl
from jax.experimental.pallas import tpu as pltpu
from jax.experimental.pallas import tpu_sc as plsc

NUM_LABELS = 1000000
EMBED_DIM = 16
BATCH = 16384
VOCAB = NUM_LABELS + 1

_NC = 2
_NS = 16
_NW = _NC * _NS
_L = 16

_W1 = 4608
_MAIN = 999936
_NCOL = _MAIN // _W1
_NCHUNKS1 = 2 * _NCOL
_ITERS1 = -(-_NCHUNKS1 // _NW)
_RSTRIDE = 1000064
_FLAT = EMBED_DIM * _RSTRIDE
_TAILW = VOCAB - _MAIN

_PER_W = BATCH // _NW
_CHUNK = 128
_NCHUNK = _PER_W // _CHUNK
_NGROUP = _PER_W // _L


def _detile_kernel(table_hbm, tail_hbm, flat_hbm, buf_a, buf_b, tail_v,
                   in_sem, out_sem, out_sem_b):
    out_sems = (out_sem, out_sem_b)
    wid = lax.axis_index("s") * _NC + lax.axis_index("c")
    bufs = (buf_a, buf_b)

    @pl.when(wid < 2)
    def _():
        a = wid
        pltpu.sync_copy(tail_hbm.at[pl.ds(a * 8, 8), :], tail_v)
        tail_copies = [
            pltpu.async_copy(
                tail_v.at[s],
                flat_hbm.at[pl.ds((a * 8 + s) * _RSTRIDE + _MAIN, _TAILW)],
                out_sem,
            )
            for s in range(8)
        ]
        for c in tail_copies:
            c.wait()

    def chunk(i):
        k = jnp.minimum(wid + i * _NW, _NCHUNKS1 - 1)
        return k % 2, (k // 2) * _W1

    def start_in(i):
        a, c0 = chunk(i)
        return pltpu.async_copy(
            table_hbm.at[pl.ds(a * 8, 8), pl.ds(c0, _W1)],
            bufs[i % 2],
            in_sem,
        )

    def start_outs(i):
        a, c0 = chunk(i)
        return [
            pltpu.async_copy(
                bufs[i % 2].at[s],
                flat_hbm.at[pl.ds((a * 8 + s) * _RSTRIDE + c0, _W1)],
                out_sems[i % 2],
            )
            for s in range(8)
        ]

    in_d = start_in(0)
    outs_prev = None
    for i in range(_ITERS1):
        in_d.wait()
        outs_i = start_outs(i)
        if i + 1 < _ITERS1:
            if outs_prev is not None:
                for c in outs_prev:
                    c.wait()
            in_d = start_in(i + 1)
            outs_prev = outs_i
        else:
            if outs_prev is not None:
                for c in outs_prev:
                    c.wait()
            for c in outs_i:
                c.wait()


def _gather_kernel(idx_hbm, flat_hbm, out_hbm, idx_v, rows_v, sem, sem_b):
    wid = lax.axis_index("s") * _NC + lax.axis_index("c")
    base = wid * _PER_W
    sems = (sem, sem_b)
    pltpu.sync_copy(idx_hbm.at[pl.ds(base, _PER_W)], idx_v)

    prev = None
    for j in range(_NCHUNK):
        off = j * _CHUNK
        cur = [
            pltpu.async_copy(
                flat_hbm.at[pl.ds(d * _RSTRIDE, _RSTRIDE)]
                .at[idx_v.at[pl.ds(off, _CHUNK)]],
                rows_v.at[d, pl.ds(off, _CHUNK)],
                sems[j % 2],
            )
            for d in range(EMBED_DIM)
        ]
        if prev is not None:
            for c in prev:
                c.wait()
        prev = cur
    for c in prev:
        c.wait()
    pltpu.sync_copy(rows_v, out_hbm.at[:, pl.ds(base, _PER_W)])


@jax.jit
def kernel(labels, label_embs):
    idx = labels.astype(jnp.int32)
    table_t = label_embs.T
    tail = label_embs[_MAIN:, :].T
    mesh = plsc.VectorSubcoreMesh(core_axis_name="c", subcore_axis_name="s")

    detile = functools.partial(
        pl.kernel,
        mesh=mesh,
        out_type=jax.ShapeDtypeStruct((_FLAT,), jnp.float32),
        scratch_types=[
            pltpu.VMEM((8, _W1), jnp.float32),
            pltpu.VMEM((8, _W1), jnp.float32),
            pltpu.VMEM((8, _TAILW), jnp.float32),
            pltpu.SemaphoreType.DMA,
            pltpu.SemaphoreType.DMA,
            pltpu.SemaphoreType.DMA,
        ],
    )(_detile_kernel)
    flat = detile(table_t, tail)

    gather = functools.partial(
        pl.kernel,
        mesh=mesh,
        out_type=jax.ShapeDtypeStruct((EMBED_DIM, BATCH), jnp.float32),
        scratch_types=[
            pltpu.VMEM((_PER_W,), jnp.int32),
            pltpu.VMEM((EMBED_DIM, _PER_W), jnp.float32),
            pltpu.SemaphoreType.DMA,
            pltpu.SemaphoreType.DMA,
        ],
        compiler_params=pltpu.CompilerParams(use_tc_tiling_on_sc=False),
    )(_gather_kernel)
    out_t = gather(idx, flat)
    return out_t.T
